# baseline (speedup 1.0000x reference)
import functools

import jax
import jax.numpy as jnp
from jax import lax
from jax.experimental import pallas as pl
from jax.experimental.pallas import tpu as pltpu
from jax.experimental.pallas import tpu_sc as plsc

N = 10000
F = 128
H = 128
C = 40
C_PAD = 48

NC = 2
NS = 16
NW = NC * NS
UNIT = 128
GROUP = 16
N_GROUPS = 5
NG_ALL = 10
EW = UNIT * GROUP * N_GROUPS
EPAD = EW * NW
NROW = 10240
ROWS_PER_TEC = NROW // NS
DUMP_ROW = 10200

def _mesh():
    return plsc.VectorSubcoreMesh(
        core_axis_name="c", subcore_axis_name="s", num_cores=NC, num_subcores=NS
    )


def _zero_rows(buf, nrows, d):
    z = jnp.zeros((16,), jnp.float32)

    def body(i, carry):
        for j in range(d // 16):
            buf[i, pl.ds(j * 16, 16)] = z
        return carry

    lax.fori_loop(0, nrows, body, 0)


def _make_deg_kernel():
    @functools.partial(
        pl.kernel,
        out_type=jax.ShapeDtypeStruct((NC, NROW, 16), jnp.float32),
        mesh=_mesh(),
        scratch_types=[
            pltpu.VMEM((4, 128), jnp.int32),
            pltpu.VMEM((128, 16), jnp.float32),
            pltpu.VMEM((ROWS_PER_TEC, 16), jnp.float32),
            pltpu.VMEM_SHARED((NROW, 16), jnp.float32),
        ],
        compiler_params=pltpu.CompilerParams(use_tc_tiling_on_sc=False),
    )
    def deg_kernel(dst_hbm, out_hbm, didx, ones_v, buf_v, acc):
        c = lax.axis_index("c")
        s = lax.axis_index("s")
        wid = c * NS + s

        one = jnp.ones((16,), jnp.float32)

        def fill_ones(i, carry):
            ones_v[i, pl.ds(0, 16)] = one
            return carry

        lax.fori_loop(0, 128, fill_ones, 0)
        _zero_rows(buf_v, ROWS_PER_TEC, 16)
        pltpu.sync_copy(buf_v, acc.at[pl.ds(s * ROWS_PER_TEC, ROWS_PER_TEC)])
        plsc.subcore_barrier()

        idx_row_base = wid * (EW // 128)

        def chunk(i, carry):
            r0 = idx_row_base + i * 4
            pltpu.sync_copy(dst_hbm.at[pl.ds(r0, 4)], didx)
            for j in range(4):
                pltpu.sync_copy(ones_v, acc.at[didx.at[j]], add=True)
            return carry

        lax.fori_loop(0, EW // 512, chunk, 0)
        plsc.subcore_barrier()

        r0 = s * ROWS_PER_TEC
        pltpu.sync_copy(acc.at[pl.ds(r0, ROWS_PER_TEC)], buf_v)
        pltpu.sync_copy(buf_v, out_hbm.at[c, pl.ds(r0, ROWS_PER_TEC)])

    return deg_kernel


def _make_scatter_kernel(d):
    @functools.partial(
        pl.kernel,
        out_type=jax.ShapeDtypeStruct((NROW, d), jnp.float32),
        mesh=_mesh(),
        scratch_types=[
            pltpu.VMEM((GROUP, 128), jnp.int32),
            pltpu.VMEM((GROUP, 128), jnp.int32),
            pltpu.VMEM((2, UNIT, d), jnp.float32),
            pltpu.VMEM_SHARED((NROW, d), jnp.float32),
            pltpu.SemaphoreType.DMA,
            pltpu.SemaphoreType.DMA,
        ],
        compiler_params=pltpu.CompilerParams(use_tc_tiling_on_sc=False),
    )
    def scatter_kernel(y_hbm, src_hbm, dst_hbm, out_hbm, sidx, didx, rows, acc, s0, s1):
        c = lax.axis_index("c")
        s = lax.axis_index("s")
        sems = (s0, s1)

        @pl.when(c == 0)
        def _core0_body():
            z = jnp.zeros((16,), jnp.float32)

            def zbody(i, carry):
                for j in range(d // 16):
                    rows[0, i, pl.ds(j * 16, 16)] = z
                return carry

            lax.fori_loop(0, UNIT, zbody, 0)
            r0 = s * ROWS_PER_TEC
            for off in range(0, ROWS_PER_TEC, UNIT):
                pltpu.sync_copy(rows.at[0], acc.at[pl.ds(r0 + off, UNIT)])
            plsc.subcore_barrier()

            idx_row_base = s * (NG_ALL * GROUP)

            def group_body(g, carry):
                rr = idx_row_base + g * GROUP
                pltpu.sync_copy(src_hbm.at[pl.ds(rr, GROUP)], sidx)
                pltpu.sync_copy(dst_hbm.at[pl.ds(rr, GROUP)], didx)
                descs = {}
                descs[0] = pltpu.async_copy(y_hbm.at[sidx.at[0]], rows.at[0], s0)
                for u in range(GROUP):
                    b = u % 2
                    descs[b].wait()
                    if u + 1 < GROUP:
                        nb = (u + 1) % 2
                        descs[nb] = pltpu.async_copy(
                            y_hbm.at[sidx.at[u + 1]], rows.at[nb], sems[nb]
                        )
                    pltpu.sync_copy(rows.at[b], acc.at[didx.at[u]], add=True)
                return carry

            lax.fori_loop(0, NG_ALL, group_body, 0)
            plsc.subcore_barrier()

            for off in range(0, ROWS_PER_TEC, UNIT):
                pltpu.sync_copy(acc.at[pl.ds(r0 + off, UNIT)], rows.at[0])
                pltpu.sync_copy(rows.at[0], out_hbm.at[pl.ds(r0 + off, UNIT)])

    return scatter_kernel


_SC_CACHE = {}


def _sc_kernels():
    if not _SC_CACHE:
        _SC_CACHE["deg"] = _make_deg_kernel()
        _SC_CACHE["sh"] = _make_scatter_kernel(H)
        _SC_CACHE["sc"] = _make_scatter_kernel(C_PAD)
    return _SC_CACHE["deg"], _SC_CACHE["sh"], _SC_CACHE["sc"]


def _tc1_body(degp, x, w1, dinv_ref, y1_ref):
    d = degp[0] + degp[1]
    deg = d[:N, 0:1] + 1.0
    dinv = 1.0 / jnp.sqrt(deg)
    xw = jnp.dot(x[...], w1[...], preferred_element_type=jnp.float32)
    dinv_ref[...] = dinv
    y1_ref[...] = xw * dinv


def _tc2_body(s1p, y1, dinv_ref, w2, b1, h1_ref, y2_ref):
    s1 = s1p[:N, :]
    dinv = dinv_ref[...]
    h1 = jnp.maximum(dinv * (s1 + y1[...]) + b1[...], 0.0)
    h1_ref[...] = h1
    y2_ref[...] = jnp.dot(h1, w2[...], preferred_element_type=jnp.float32) * dinv


def _tc3_body(s2p, y2, dinv_ref, x, h1, w3p, b2, y3_ref):
    s2 = s2p[:N, :]
    dinv = dinv_ref[...]
    h2 = jnp.maximum(dinv * (s2 + y2[...]) + b2[...], 0.0)
    xw3 = (
        jnp.dot(x[...], w3p[0:F, :], preferred_element_type=jnp.float32)
        + jnp.dot(h1[...], w3p[F:F + H, :], preferred_element_type=jnp.float32)
        + jnp.dot(h2, w3p[F + H:F + 2 * H, :], preferred_element_type=jnp.float32)
    )
    y3_ref[...] = xw3 * dinv


def _tc4_body(s3p, y3, dinv_ref, b3p, out_ref):
    s3 = s3p[:N, :]
    o = jnp.maximum(dinv_ref[...] * (s3 + y3[...]) + b3p[...], 0.0)
    out_ref[...] = o[:, :C]


_tc1 = pl.pallas_call(
    _tc1_body,
    out_shape=(
        jax.ShapeDtypeStruct((N, 1), jnp.float32),
        jax.ShapeDtypeStruct((N, H), jnp.float32),
    ),
)

_tc2 = pl.pallas_call(
    _tc2_body,
    out_shape=(
        jax.ShapeDtypeStruct((N, H), jnp.float32),
        jax.ShapeDtypeStruct((N, H), jnp.float32),
    ),
)

_tc3 = pl.pallas_call(
    _tc3_body,
    out_shape=jax.ShapeDtypeStruct((N, C_PAD), jnp.float32),
)

_tc4 = pl.pallas_call(
    _tc4_body,
    out_shape=jax.ShapeDtypeStruct((N, C), jnp.float32),
)


def kernel(x, edge_index, W1, b1, W2, b2, W3, b3):
    e = edge_index.shape[1]
    pad = EPAD - e
    src = jnp.concatenate(
        [edge_index[0], jnp.zeros((pad,), edge_index.dtype)]
    ).reshape(EPAD // 128, 128)
    dst = jnp.concatenate(
        [edge_index[1], jnp.full((pad,), DUMP_ROW, edge_index.dtype)]
    ).reshape(EPAD // 128, 128)

    w3p = jnp.pad(W3, ((0, 0), (0, C_PAD - C)))
    b1r = b1.reshape(1, H)
    b2r = b2.reshape(1, H)
    b3r = jnp.pad(b3, (0, C_PAD - C)).reshape(1, C_PAD)

    deg_sc, scatter_h, scatter_c = _sc_kernels()
    degp = deg_sc(dst)
    dinv, y1 = _tc1(degp, x, W1)
    s1p = scatter_h(y1, src, dst)
    h1, y2 = _tc2(s1p, y1, dinv, W2, b1r)
    s2p = scatter_h(y2, src, dst)
    y3 = _tc3(s2p, y2, dinv, x, h1, w3p, b2r)
    s3p = scatter_c(y3, src, dst)
    out = _tc4(s3p, y3, dinv, b3r)
    return out


# speedup vs baseline: 1.5039x; 1.5039x over previous
import functools

import jax
import jax.numpy as jnp
from jax import lax
from jax.experimental import pallas as pl
from jax.experimental.pallas import tpu as pltpu
from jax.experimental.pallas import tpu_sc as plsc

N = 10000
F = 128
H = 128
C = 40
C_PAD = 48

NC = 2
NS = 16
NW = NC * NS
UNIT = 128
GROUP = 16
N_GROUPS = 5
EW = UNIT * GROUP * N_GROUPS
EPAD = EW * NW
NROW = 10240
ROWS_PER_TEC = NROW // NS
DUMP_ROW = 10200

def _mesh():
    return plsc.VectorSubcoreMesh(
        core_axis_name="c", subcore_axis_name="s", num_cores=NC, num_subcores=NS
    )


def _zero_rows(buf, nrows, d):
    z = jnp.zeros((16,), jnp.float32)

    def body(i, carry):
        for j in range(d // 16):
            buf[i, pl.ds(j * 16, 16)] = z
        return carry

    lax.fori_loop(0, nrows, body, 0)


def _make_deg_kernel():
    @functools.partial(
        pl.kernel,
        out_type=jax.ShapeDtypeStruct((NC, NROW, 16), jnp.float32),
        mesh=_mesh(),
        scratch_types=[
            pltpu.VMEM((4, 128), jnp.int32),
            pltpu.VMEM((128, 16), jnp.float32),
            pltpu.VMEM((ROWS_PER_TEC, 16), jnp.float32),
            pltpu.VMEM_SHARED((NROW, 16), jnp.float32),
        ],
        compiler_params=pltpu.CompilerParams(use_tc_tiling_on_sc=False),
    )
    def deg_kernel(dst_hbm, out_hbm, didx, ones_v, buf_v, acc):
        c = lax.axis_index("c")
        s = lax.axis_index("s")
        wid = c * NS + s

        one = jnp.ones((16,), jnp.float32)

        def fill_ones(i, carry):
            ones_v[i, pl.ds(0, 16)] = one
            return carry

        lax.fori_loop(0, 128, fill_ones, 0)
        _zero_rows(buf_v, ROWS_PER_TEC, 16)
        pltpu.sync_copy(buf_v, acc.at[pl.ds(s * ROWS_PER_TEC, ROWS_PER_TEC)])
        plsc.subcore_barrier()

        idx_row_base = wid * (EW // 128)

        def chunk(i, carry):
            r0 = idx_row_base + i * 4
            pltpu.sync_copy(dst_hbm.at[pl.ds(r0, 4)], didx)
            for j in range(4):
                pltpu.sync_copy(ones_v, acc.at[didx.at[j]], add=True)
            return carry

        lax.fori_loop(0, EW // 512, chunk, 0)
        plsc.subcore_barrier()

        r0 = s * ROWS_PER_TEC
        pltpu.sync_copy(acc.at[pl.ds(r0, ROWS_PER_TEC)], buf_v)
        pltpu.sync_copy(buf_v, out_hbm.at[c, pl.ds(r0, ROWS_PER_TEC)])

    return deg_kernel


def _make_scatter_kernel(d, g0):
    g1 = 10 - g0

    @functools.partial(
        pl.kernel,
        out_type=jax.ShapeDtypeStruct((NC, NROW, d), jnp.float32),
        mesh=_mesh(),
        scratch_types=[
            pltpu.VMEM((GROUP, 128), jnp.int32),
            pltpu.VMEM((GROUP, 128), jnp.int32),
            pltpu.VMEM((2, UNIT, d), jnp.float32),
            pltpu.VMEM_SHARED((NROW, d), jnp.float32),
            pltpu.SemaphoreType.DMA,
            pltpu.SemaphoreType.DMA,
        ],
        compiler_params=pltpu.CompilerParams(use_tc_tiling_on_sc=False),
    )
    def scatter_kernel(y_hbm, src_hbm, dst_hbm, out_hbm, sidx, didx, rows, acc, s0, s1):
        c = lax.axis_index("c")
        s = lax.axis_index("s")
        sems = (s0, s1)

        z = jnp.zeros((16,), jnp.float32)

        def zbody(i, carry):
            for j in range(d // 16):
                rows[0, i, pl.ds(j * 16, 16)] = z
            return carry

        lax.fori_loop(0, UNIT, zbody, 0)
        r0 = s * ROWS_PER_TEC
        for off in range(0, ROWS_PER_TEC, UNIT):
            pltpu.sync_copy(rows.at[0], acc.at[pl.ds(r0 + off, UNIT)])
        plsc.subcore_barrier()

        ngroups = jnp.where(c == 0, g0, g1)
        idx_row_base = jnp.where(
            c == 0, s * (g0 * GROUP), NS * g0 * GROUP + s * (g1 * GROUP)
        )

        def group_body(g, carry):
            rr = idx_row_base + g * GROUP
            pltpu.sync_copy(src_hbm.at[pl.ds(rr, GROUP)], sidx)
            pltpu.sync_copy(dst_hbm.at[pl.ds(rr, GROUP)], didx)
            descs = {}
            descs[0] = pltpu.async_copy(y_hbm.at[sidx.at[0]], rows.at[0], s0)
            for u in range(GROUP):
                b = u % 2
                descs[b].wait()
                if u + 1 < GROUP:
                    nb = (u + 1) % 2
                    descs[nb] = pltpu.async_copy(
                        y_hbm.at[sidx.at[u + 1]], rows.at[nb], sems[nb]
                    )
                pltpu.sync_copy(rows.at[b], acc.at[didx.at[u]], add=True)
            return carry

        lax.fori_loop(0, ngroups, group_body, 0)
        plsc.subcore_barrier()

        for off in range(0, ROWS_PER_TEC, UNIT):
            pltpu.sync_copy(acc.at[pl.ds(r0 + off, UNIT)], rows.at[0])
            pltpu.sync_copy(rows.at[0], out_hbm.at[c, pl.ds(r0 + off, UNIT)])

    return scatter_kernel


_SC_CACHE = {}


def _sc_kernels():
    if not _SC_CACHE:
        _SC_CACHE["deg"] = _make_deg_kernel()
        _SC_CACHE["sh"] = _make_scatter_kernel(H, 9)
        _SC_CACHE["sc"] = _make_scatter_kernel(C_PAD, 8)
    return _SC_CACHE["deg"], _SC_CACHE["sh"], _SC_CACHE["sc"]


def _tc1_body(degp, x, w1, dinv_ref, y1_ref):
    d = degp[0] + degp[1]
    deg = d[:N, 0:1] + 1.0
    dinv = 1.0 / jnp.sqrt(deg)
    xw = jnp.dot(x[...], w1[...], preferred_element_type=jnp.float32)
    dinv_ref[...] = dinv
    y1_ref[...] = xw * dinv


def _tc2_body(s1p, y1, dinv_ref, w2, b1, h1_ref, y2_ref):
    s1 = s1p[0, :N, :] + s1p[1, :N, :]
    dinv = dinv_ref[...]
    h1 = jnp.maximum(dinv * (s1 + y1[...]) + b1[...], 0.0)
    h1_ref[...] = h1
    y2_ref[...] = jnp.dot(h1, w2[...], preferred_element_type=jnp.float32) * dinv


def _tc3_body(s2p, y2, dinv_ref, x, h1, w3p, b2, y3_ref):
    s2 = s2p[0, :N, :] + s2p[1, :N, :]
    dinv = dinv_ref[...]
    h2 = jnp.maximum(dinv * (s2 + y2[...]) + b2[...], 0.0)
    xw3 = (
        jnp.dot(x[...], w3p[0:F, :], preferred_element_type=jnp.float32)
        + jnp.dot(h1[...], w3p[F:F + H, :], preferred_element_type=jnp.float32)
        + jnp.dot(h2, w3p[F + H:F + 2 * H, :], preferred_element_type=jnp.float32)
    )
    y3_ref[...] = xw3 * dinv


def _tc4_body(s3p, y3, dinv_ref, b3p, out_ref):
    s3 = s3p[0, :N, :] + s3p[1, :N, :]
    o = jnp.maximum(dinv_ref[...] * (s3 + y3[...]) + b3p[...], 0.0)
    out_ref[...] = o[:, :C]


_tc1 = pl.pallas_call(
    _tc1_body,
    out_shape=(
        jax.ShapeDtypeStruct((N, 1), jnp.float32),
        jax.ShapeDtypeStruct((N, H), jnp.float32),
    ),
)

_tc2 = pl.pallas_call(
    _tc2_body,
    out_shape=(
        jax.ShapeDtypeStruct((N, H), jnp.float32),
        jax.ShapeDtypeStruct((N, H), jnp.float32),
    ),
)

_tc3 = pl.pallas_call(
    _tc3_body,
    out_shape=jax.ShapeDtypeStruct((N, C_PAD), jnp.float32),
)

_tc4 = pl.pallas_call(
    _tc4_body,
    out_shape=jax.ShapeDtypeStruct((N, C), jnp.float32),
)


def kernel(x, edge_index, W1, b1, W2, b2, W3, b3):
    e = edge_index.shape[1]
    pad = EPAD - e
    src = jnp.concatenate(
        [edge_index[0], jnp.zeros((pad,), edge_index.dtype)]
    ).reshape(EPAD // 128, 128)
    dst = jnp.concatenate(
        [edge_index[1], jnp.full((pad,), DUMP_ROW, edge_index.dtype)]
    ).reshape(EPAD // 128, 128)

    w3p = jnp.pad(W3, ((0, 0), (0, C_PAD - C)))
    b1r = b1.reshape(1, H)
    b2r = b2.reshape(1, H)
    b3r = jnp.pad(b3, (0, C_PAD - C)).reshape(1, C_PAD)

    deg_sc, scatter_h, scatter_c = _sc_kernels()
    degp = deg_sc(dst)
    dinv, y1 = _tc1(degp, x, W1)
    s1p = scatter_h(y1, src, dst)
    h1, y2 = _tc2(s1p, y1, dinv, W2, b1r)
    s2p = scatter_h(y2, src, dst)
    y3 = _tc3(s2p, y2, dinv, x, h1, w3p, b2r)
    s3p = scatter_c(y3, src, dst)
    out = _tc4(s3p, y3, dinv, b3r)
    return out


# speedup vs baseline: 1.5363x; 1.0215x over previous
import functools

import jax
import jax.numpy as jnp
from jax import lax
from jax.experimental import pallas as pl
from jax.experimental.pallas import tpu as pltpu
from jax.experimental.pallas import tpu_sc as plsc

N = 10000
F = 128
H = 128
C = 40
C_PAD = 48

NC = 2
NS = 16
NW = NC * NS
UNIT = 128
GROUP = 16
N_GROUPS = 5
EW = UNIT * GROUP * N_GROUPS
EPAD = EW * NW
NROW = 10240
ROWS_PER_TEC = NROW // NS
DUMP_ROW = 10200

def _mesh():
    return plsc.VectorSubcoreMesh(
        core_axis_name="c", subcore_axis_name="s", num_cores=NC, num_subcores=NS
    )


def _zero_rows(buf, nrows, d):
    z = jnp.zeros((16,), jnp.float32)

    def body(i, carry):
        for j in range(d // 16):
            buf[i, pl.ds(j * 16, 16)] = z
        return carry

    lax.fori_loop(0, nrows, body, 0)


def _make_deg_kernel():
    @functools.partial(
        pl.kernel,
        out_type=jax.ShapeDtypeStruct((NC, NROW, 16), jnp.float32),
        mesh=_mesh(),
        scratch_types=[
            pltpu.VMEM((4, 128), jnp.int32),
            pltpu.VMEM((128, 16), jnp.float32),
            pltpu.VMEM((ROWS_PER_TEC, 16), jnp.float32),
            pltpu.VMEM_SHARED((NROW, 16), jnp.float32),
        ],
        compiler_params=pltpu.CompilerParams(use_tc_tiling_on_sc=False),
    )
    def deg_kernel(dst_hbm, out_hbm, didx, ones_v, buf_v, acc):
        c = lax.axis_index("c")
        s = lax.axis_index("s")
        wid = c * NS + s

        one = jnp.ones((16,), jnp.float32)

        def fill_ones(i, carry):
            ones_v[i, pl.ds(0, 16)] = one
            return carry

        lax.fori_loop(0, 128, fill_ones, 0)
        _zero_rows(buf_v, ROWS_PER_TEC, 16)
        pltpu.sync_copy(buf_v, acc.at[pl.ds(s * ROWS_PER_TEC, ROWS_PER_TEC)])
        plsc.subcore_barrier()

        idx_row_base = wid * (EW // 128)

        def chunk(i, carry):
            r0 = idx_row_base + i * 4
            pltpu.sync_copy(dst_hbm.at[pl.ds(r0, 4)], didx)
            for j in range(4):
                pltpu.sync_copy(ones_v, acc.at[didx.at[j]], add=True)
            return carry

        lax.fori_loop(0, EW // 512, chunk, 0)
        plsc.subcore_barrier()

        r0 = s * ROWS_PER_TEC
        pltpu.sync_copy(acc.at[pl.ds(r0, ROWS_PER_TEC)], buf_v)
        pltpu.sync_copy(buf_v, out_hbm.at[c, pl.ds(r0, ROWS_PER_TEC)])

    return deg_kernel


def _make_scatter_kernel(d, g0):
    g1 = 10 - g0

    @functools.partial(
        pl.kernel,
        out_type=jax.ShapeDtypeStruct((NC, NROW, d), jnp.float32),
        mesh=_mesh(),
        scratch_types=[
            pltpu.VMEM((GROUP, 128), jnp.int32),
            pltpu.VMEM((GROUP, 128), jnp.int32),
            pltpu.VMEM((2, UNIT, d), jnp.float32),
            pltpu.VMEM_SHARED((NROW, d), jnp.float32),
            pltpu.SemaphoreType.DMA,
            pltpu.SemaphoreType.DMA,
        ],
        compiler_params=pltpu.CompilerParams(use_tc_tiling_on_sc=False),
    )
    def scatter_kernel(y_hbm, src_hbm, dst_hbm, out_hbm, sidx, didx, rows, acc, s0, s1):
        c = lax.axis_index("c")
        s = lax.axis_index("s")
        sems = (s0, s1)

        z = jnp.zeros((16,), jnp.float32)

        def zbody(i, carry):
            for j in range(d // 16):
                rows[0, i, pl.ds(j * 16, 16)] = z
            return carry

        lax.fori_loop(0, UNIT, zbody, 0)
        r0 = s * ROWS_PER_TEC
        for off in range(0, ROWS_PER_TEC, UNIT):
            pltpu.sync_copy(rows.at[0], acc.at[pl.ds(r0 + off, UNIT)])
        plsc.subcore_barrier()

        ngroups = jnp.where(c == 0, g0, g1)
        idx_row_base = jnp.where(
            c == 0, s * (g0 * GROUP), NS * g0 * GROUP + s * (g1 * GROUP)
        )

        def group_body(g, carry):
            rr = idx_row_base + g * GROUP
            pltpu.sync_copy(src_hbm.at[pl.ds(rr, GROUP)], sidx)
            pltpu.sync_copy(dst_hbm.at[pl.ds(rr, GROUP)], didx)
            descs = {
                0: pltpu.async_copy(y_hbm.at[sidx.at[0]], rows.at[0], s0),
                1: pltpu.async_copy(y_hbm.at[sidx.at[1]], rows.at[1], s1),
            }
            for u in range(GROUP):
                b = u % 2
                descs[b].wait()
                pltpu.sync_copy(rows.at[b], acc.at[didx.at[u]], add=True)
                if u + 2 < GROUP:
                    descs[b] = pltpu.async_copy(
                        y_hbm.at[sidx.at[u + 2]], rows.at[b], sems[b]
                    )
            return carry

        lax.fori_loop(0, ngroups, group_body, 0)
        plsc.subcore_barrier()

        for off in range(0, ROWS_PER_TEC, UNIT):
            pltpu.sync_copy(acc.at[pl.ds(r0 + off, UNIT)], rows.at[0])
            pltpu.sync_copy(rows.at[0], out_hbm.at[c, pl.ds(r0 + off, UNIT)])

    return scatter_kernel


_SC_CACHE = {}


def _sc_kernels():
    if not _SC_CACHE:
        _SC_CACHE["deg"] = _make_deg_kernel()
        _SC_CACHE["sh"] = _make_scatter_kernel(H, 9)
        _SC_CACHE["sc"] = _make_scatter_kernel(C_PAD, 8)
    return _SC_CACHE["deg"], _SC_CACHE["sh"], _SC_CACHE["sc"]


def _tc1_body(degp, x, w1, dinv_ref, y1_ref):
    d = degp[0] + degp[1]
    deg = d[:N, 0:1] + 1.0
    dinv = 1.0 / jnp.sqrt(deg)
    xw = jnp.dot(x[...], w1[...], preferred_element_type=jnp.float32)
    dinv_ref[...] = dinv
    y1_ref[...] = xw * dinv


def _tc2_body(s1p, y1, dinv_ref, w2, b1, h1_ref, y2_ref):
    s1 = s1p[0, :N, :] + s1p[1, :N, :]
    dinv = dinv_ref[...]
    h1 = jnp.maximum(dinv * (s1 + y1[...]) + b1[...], 0.0)
    h1_ref[...] = h1
    y2_ref[...] = jnp.dot(h1, w2[...], preferred_element_type=jnp.float32) * dinv


def _tc3_body(s2p, y2, dinv_ref, x, h1, w3p, b2, y3_ref):
    s2 = s2p[0, :N, :] + s2p[1, :N, :]
    dinv = dinv_ref[...]
    h2 = jnp.maximum(dinv * (s2 + y2[...]) + b2[...], 0.0)
    xw3 = (
        jnp.dot(x[...], w3p[0:F, :], preferred_element_type=jnp.float32)
        + jnp.dot(h1[...], w3p[F:F + H, :], preferred_element_type=jnp.float32)
        + jnp.dot(h2, w3p[F + H:F + 2 * H, :], preferred_element_type=jnp.float32)
    )
    y3_ref[...] = xw3 * dinv


def _tc4_body(s3p, y3, dinv_ref, b3p, out_ref):
    s3 = s3p[0, :N, :] + s3p[1, :N, :]
    o = jnp.maximum(dinv_ref[...] * (s3 + y3[...]) + b3p[...], 0.0)
    out_ref[...] = o[:, :C]


_tc1 = pl.pallas_call(
    _tc1_body,
    out_shape=(
        jax.ShapeDtypeStruct((N, 1), jnp.float32),
        jax.ShapeDtypeStruct((N, H), jnp.float32),
    ),
)

_tc2 = pl.pallas_call(
    _tc2_body,
    out_shape=(
        jax.ShapeDtypeStruct((N, H), jnp.float32),
        jax.ShapeDtypeStruct((N, H), jnp.float32),
    ),
)

_tc3 = pl.pallas_call(
    _tc3_body,
    out_shape=jax.ShapeDtypeStruct((N, C_PAD), jnp.float32),
)

_tc4 = pl.pallas_call(
    _tc4_body,
    out_shape=jax.ShapeDtypeStruct((N, C), jnp.float32),
)


def kernel(x, edge_index, W1, b1, W2, b2, W3, b3):
    e = edge_index.shape[1]
    pad = EPAD - e
    src = jnp.concatenate(
        [edge_index[0], jnp.zeros((pad,), edge_index.dtype)]
    ).reshape(EPAD // 128, 128)
    dst = jnp.concatenate(
        [edge_index[1], jnp.full((pad,), DUMP_ROW, edge_index.dtype)]
    ).reshape(EPAD // 128, 128)

    w3p = jnp.pad(W3, ((0, 0), (0, C_PAD - C)))
    b1r = b1.reshape(1, H)
    b2r = b2.reshape(1, H)
    b3r = jnp.pad(b3, (0, C_PAD - C)).reshape(1, C_PAD)

    deg_sc, scatter_h, scatter_c = _sc_kernels()
    degp = deg_sc(dst)
    dinv, y1 = _tc1(degp, x, W1)
    s1p = scatter_h(y1, src, dst)
    h1, y2 = _tc2(s1p, y1, dinv, W2, b1r)
    s2p = scatter_h(y2, src, dst)
    y3 = _tc3(s2p, y2, dinv, x, h1, w3p, b2r)
    s3p = scatter_c(y3, src, dst)
    out = _tc4(s3p, y3, dinv, b3r)
    return out
